# baseline (speedup 1.0000x reference)
import functools

import jax
import jax.numpy as jnp
from jax import lax
from jax.experimental import pallas as pl
from jax.experimental.pallas import tpu as pltpu
from jax.experimental.pallas import tpu_sc as plsc


def _argmin_kernel(z_ref, embn2_ref, idx_ref):
    z = z_ref[0]
    embn2 = embn2_ref[...]

    e_sq = 0.25 * jnp.sum(embn2 * embn2, axis=1, keepdims=True)
    scores = jax.lax.dot_general(
        embn2, z, (((1,), (0,)), ((), ())),
        preferred_element_type=jnp.float32)
    d2 = scores + e_sq
    idx_ref[0] = jnp.argmin(d2, axis=0)[None, :]


def _sc_gather_kernel(c_dim, t_dim, embt_hbm, idx_hbm, out_hbm,
                      et_v, idx_v, out_v):
    nc = lax.axis_size("c")
    wid = lax.axis_index("s") * nc + lax.axis_index("c")
    pltpu.sync_copy(embt_hbm, et_v)
    pltpu.sync_copy(idx_hbm.at[pl.ds(wid * t_dim, t_dim)], idx_v)

    def body(j, carry):
        iv = idx_v[pl.ds(j * 16, 16)]
        for c in range(c_dim):
            cv = jnp.full((16,), c, jnp.int32)
            out_v[c, pl.ds(j * 16, 16)] = plsc.load_gather(et_v, [cv, iv])
        return carry

    lax.fori_loop(0, t_dim // 16, body, 0)
    pltpu.sync_copy(out_v, out_hbm.at[wid])


def kernel(z_e, embedding):
    B, C, H, W = z_e.shape
    K = embedding.shape[0]
    T = H * W
    z = z_e.reshape(B, C, T)

    idx3 = pl.pallas_call(
        _argmin_kernel,
        grid=(B,),
        in_specs=[
            pl.BlockSpec((1, C, T), lambda b: (b, 0, 0)),
            pl.BlockSpec((K, C), lambda b: (0, 0)),
        ],
        out_specs=pl.BlockSpec((1, 1, T), lambda b: (b, 0, 0)),
        out_shape=jax.ShapeDtypeStruct((B, 1, T), jnp.int32),
    )(z, -2.0 * embedding)

    mesh = plsc.VectorSubcoreMesh(core_axis_name="c", subcore_axis_name="s")
    sc_gather = functools.partial(
        pl.kernel,
        mesh=mesh,
        compiler_params=pltpu.CompilerParams(needs_layout_passes=False),
        out_type=jax.ShapeDtypeStruct((B, C, T), jnp.float32),
        scratch_types=[
            pltpu.VMEM((C, K), jnp.float32),
            pltpu.VMEM((T,), jnp.int32),
            pltpu.VMEM((C, T), jnp.float32),
        ],
    )(functools.partial(_sc_gather_kernel, C, T))

    out = sc_gather(embedding.T, idx3.reshape(B * T))
    return out.reshape(B, C, H, W)


# speedup vs baseline: 1.1596x; 1.1596x over previous
import functools

import jax
import jax.numpy as jnp
from jax import lax
from jax.experimental import pallas as pl
from jax.experimental.pallas import tpu as pltpu
from jax.experimental.pallas import tpu_sc as plsc


def _argmin_kernel(z_ref, embn2_ref, idx_ref):
    z = z_ref[0]
    embn2 = embn2_ref[...]

    e_sq = 0.25 * jnp.sum(embn2 * embn2, axis=1, keepdims=True)
    scores = jax.lax.dot_general(
        embn2, z, (((1,), (0,)), ((), ())),
        preferred_element_type=jnp.float32)
    d2 = scores + e_sq
    idx_ref[0] = jnp.argmin(d2, axis=0)[None, :]


def _sc_gather_kernel(c_dim, k_dim, t_dim, embt_hbm, idx_hbm, out_hbm,
                      et_v, idx_v, out_v):
    nc = lax.axis_size("c")
    wid = lax.axis_index("s") * nc + lax.axis_index("c")
    pltpu.sync_copy(embt_hbm, et_v)
    pltpu.sync_copy(idx_hbm.at[pl.ds(wid * t_dim, t_dim)], idx_v)

    @plsc.parallel_loop(0, t_dim // 16, unroll=2)
    def body(j):
        iv = idx_v[pl.ds(j * 16, 16)]
        for c in range(c_dim):
            idxc = iv + jnp.int32(c * k_dim)
            out_v[c, pl.ds(j * 16, 16)] = plsc.load_gather(et_v, [idxc])

    pltpu.sync_copy(out_v, out_hbm.at[wid])


def kernel(z_e, embedding):
    B, C, H, W = z_e.shape
    K = embedding.shape[0]
    T = H * W
    z = z_e.reshape(B, C, T)

    idx3 = pl.pallas_call(
        _argmin_kernel,
        grid=(B,),
        in_specs=[
            pl.BlockSpec((1, C, T), lambda b: (b, 0, 0)),
            pl.BlockSpec((K, C), lambda b: (0, 0)),
        ],
        out_specs=pl.BlockSpec((1, 1, T), lambda b: (b, 0, 0)),
        out_shape=jax.ShapeDtypeStruct((B, 1, T), jnp.int32),
    )(z, -2.0 * embedding)

    mesh = plsc.VectorSubcoreMesh(core_axis_name="c", subcore_axis_name="s")
    sc_gather = functools.partial(
        pl.kernel,
        mesh=mesh,
        compiler_params=pltpu.CompilerParams(needs_layout_passes=False),
        out_type=jax.ShapeDtypeStruct((B, C, T), jnp.float32),
        scratch_types=[
            pltpu.VMEM((C * K,), jnp.float32),
            pltpu.VMEM((T,), jnp.int32),
            pltpu.VMEM((C, T), jnp.float32),
        ],
    )(functools.partial(_sc_gather_kernel, C, K, T))

    out = sc_gather(embedding.T.reshape(C * K), idx3.reshape(B * T))
    return out.reshape(B, C, H, W)


# speedup vs baseline: 1.8755x; 1.6175x over previous
import jax
import jax.numpy as jnp
from jax.experimental import pallas as pl


def _vq_block_kernel(z_ref, emb_ref, embn2_ref, out_ref):
    emb = emb_ref[...]
    embn2 = embn2_ref[...]

    e_sq = jnp.sum(emb * emb, axis=1, keepdims=True)
    for b in range(z_ref.shape[0]):
        z = z_ref[b]
        scores = jax.lax.dot_general(
            embn2, z, (((1,), (0,)), ((), ())),
            preferred_element_type=jnp.float32)
        d2 = scores + e_sq

        rows = jax.lax.broadcasted_iota(jnp.int32, d2.shape, 0)
        idx = jnp.argmin(d2, axis=0)[None, :]

        onehot = (rows == idx).astype(jnp.float32)
        out_ref[b] = jax.lax.dot_general(
            emb, onehot, (((0,), (0,)), ((), ())),
            preferred_element_type=jnp.float32)


def kernel(z_e, embedding):
    B, C, H, W = z_e.shape
    K = embedding.shape[0]
    T = H * W
    z = z_e.reshape(B, C, T)
    NB = 2
    out = pl.pallas_call(
        _vq_block_kernel,
        grid=(B // NB,),
        in_specs=[
            pl.BlockSpec((NB, C, T), lambda b: (b, 0, 0)),
            pl.BlockSpec((K, C), lambda b: (0, 0)),
            pl.BlockSpec((K, C), lambda b: (0, 0)),
        ],
        out_specs=pl.BlockSpec((NB, C, T), lambda b: (b, 0, 0)),
        out_shape=jax.ShapeDtypeStruct((B, C, T), jnp.float32),
    )(z, embedding, -2.0 * embedding)
    return out.reshape(B, C, H, W)
